# Initial kernel scaffold; baseline (speedup 1.0000x reference)
import jax
import jax.numpy as jnp
from jax.experimental import pallas as pl


def kernel(x, emb_table, pos_table):
    raise NotImplementedError("write your pallas kernel here")


# speedup vs baseline: 3.2855x; 3.2855x over previous
import functools

import jax
import jax.numpy as jnp
from jax import lax
from jax.experimental import pallas as pl
from jax.experimental.pallas import tpu as pltpu
from jax.experimental.pallas import tpu_sc as plsc

_NC = 2
_NS = 16
_NW = _NC * _NS
_LANES = 16


@functools.lru_cache(maxsize=None)
def _build(n_rows: int, d: int, t: int, vocab: int):
    assert n_rows % _NW == 0
    rows_per_w = n_rows // _NW
    chunk = t
    assert rows_per_w % chunk == 0
    n_chunks = rows_per_w // chunk
    vecs_per_row = d // _LANES

    mesh = plsc.VectorSubcoreMesh(core_axis_name="c", subcore_axis_name="s")

    @functools.partial(
        pl.kernel,
        mesh=mesh,
        compiler_params=pltpu.CompilerParams(use_tc_tiling_on_sc=False),
        out_type=jax.ShapeDtypeStruct((n_rows, d), jnp.float32),
        scratch_types=[
            pltpu.VMEM((rows_per_w,), jnp.int32),
            pltpu.VMEM((t, d), jnp.float32),
            pltpu.VMEM((chunk, d), jnp.float32),
            pltpu.SemaphoreType.DMA,
        ],
    )
    def emb_kernel(idx_hbm, table_hbm, pos_hbm, out_hbm, idx_v, pos_v, rows_v, sem):
        wid = lax.axis_index("s") * _NC + lax.axis_index("c")
        base = wid * rows_per_w
        pltpu.sync_copy(idx_hbm.at[pl.ds(base, rows_per_w)], idx_v)
        pltpu.sync_copy(pos_hbm.at[pl.ds(0, t)], pos_v)

        def chunk_body(g, _):
            off = g * chunk
            pltpu.async_copy(
                table_hbm.at[idx_v.at[pl.ds(off, chunk)]], rows_v, sem
            ).wait()

            def row_body(r, _):
                for c in range(vecs_per_row):
                    sl = pl.ds(c * _LANES, _LANES)
                    rows_v[r, sl] = rows_v[r, sl] + pos_v[r, sl]
                return 0

            lax.fori_loop(0, chunk, row_body, 0)
            pltpu.sync_copy(rows_v, out_hbm.at[pl.ds(base + off, chunk)])
            return 0

        lax.fori_loop(0, n_chunks, chunk_body, 0)

    return emb_kernel


def kernel(x, emb_table, pos_table):
    b, t = x.shape
    vocab, d = emb_table.shape
    xf = x.reshape(-1).astype(jnp.int32)
    out = _build(b * t, d, t, vocab)(xf, emb_table, pos_table)
    return out.reshape(b, t, d)


# speedup vs baseline: 4.1264x; 1.2559x over previous
import functools

import jax
import jax.numpy as jnp
from jax import lax
from jax.experimental import pallas as pl
from jax.experimental.pallas import tpu as pltpu
from jax.experimental.pallas import tpu_sc as plsc

_NC = 2
_NS = 16
_NW = _NC * _NS
_LANES = 16
_NBUF = 4


@functools.lru_cache(maxsize=None)
def _build(n_rows: int, d: int, t: int, vocab: int):
    assert n_rows % _NW == 0
    rows_per_w = n_rows // _NW
    chunk = t
    assert rows_per_w % chunk == 0
    n_chunks = rows_per_w // chunk
    assert n_chunks >= 2 * _NBUF and (n_chunks - 4) % _NBUF == 0
    vecs_per_row = d // _LANES

    mesh = plsc.VectorSubcoreMesh(core_axis_name="c", subcore_axis_name="s")

    @functools.partial(
        pl.kernel,
        mesh=mesh,
        compiler_params=pltpu.CompilerParams(use_tc_tiling_on_sc=False),
        out_type=jax.ShapeDtypeStruct((n_rows, d), jnp.float32),
        scratch_types=[
            pltpu.VMEM((rows_per_w,), jnp.int32),
            pltpu.VMEM((t, d), jnp.float32),
        ]
        + [pltpu.VMEM((chunk, d), jnp.float32) for _ in range(_NBUF)]
        + [pltpu.SemaphoreType.DMA((_NBUF,)), pltpu.SemaphoreType.DMA((_NBUF,))],
    )
    def emb_kernel(idx_hbm, table_hbm, pos_hbm, out_hbm,
                   idx_v, pos_v, b0, b1, b2, b3, gsem, ssem):
        bufs = [b0, b1, b2, b3]
        wid = lax.axis_index("s") * _NC + lax.axis_index("c")
        base = wid * rows_per_w
        pltpu.sync_copy(idx_hbm.at[pl.ds(base, rows_per_w)], idx_v)
        pltpu.sync_copy(pos_hbm.at[pl.ds(0, t)], pos_v)

        def gather_desc(g, s):
            return pltpu.make_async_copy(
                table_hbm.at[idx_v.at[pl.ds(g * chunk, chunk)]],
                bufs[s], gsem.at[s])

        def store_desc(g, s):
            return pltpu.make_async_copy(
                bufs[s], out_hbm.at[pl.ds(base + g * chunk, chunk)], ssem.at[s])

        def do_add(s):
            buf = bufs[s]

            def row_body(r, _):
                for u in range(2):
                    for c in range(vecs_per_row):
                        sl = pl.ds(c * _LANES, _LANES)
                        buf[2 * r + u, sl] = buf[2 * r + u, sl] + pos_v[2 * r + u, sl]
                return 0

            lax.fori_loop(0, chunk // 2, row_body, 0)

        gather_desc(0, 0).start()
        gather_desc(1, 1).start()
        for g in (0, 1):
            gather_desc(g, g).wait()
            do_add(g)
            store_desc(g, g).start()
            gather_desc(g + 2, g + 2).start()

        def block_body(i, _):
            g0 = 2 + i * _NBUF
            for b in range(_NBUF):
                g = g0 + b
                s = (2 + b) % _NBUF
                gather_desc(g, s).wait()
                do_add(s)
                store_desc(g, s).start()
                s2 = b % _NBUF
                store_desc(g - 2, s2).wait()
                gather_desc(g + 2, s2).start()
            return 0

        lax.fori_loop(0, (n_chunks - 4) // _NBUF, block_body, 0)

        for g in (n_chunks - 2, n_chunks - 1):
            s = g % _NBUF
            gather_desc(g, s).wait()
            do_add(s)
            store_desc(g, s).start()
            store_desc(g - 2, (g - 2) % _NBUF).wait()
        for g in (n_chunks - 2, n_chunks - 1):
            store_desc(g, g % _NBUF).wait()

    return emb_kernel


def kernel(x, emb_table, pos_table):
    b, t = x.shape
    vocab, d = emb_table.shape
    xf = x.reshape(-1).astype(jnp.int32)
    out = _build(b * t, d, t, vocab)(xf, emb_table, pos_table)
    return out.reshape(b, t, d)
